# Initial kernel scaffold; baseline (speedup 1.0000x reference)
import jax
import jax.numpy as jnp
from jax.experimental import pallas as pl


def kernel(x, timestamp, bboxes, feats, w_reduce, b_reduce, w_num, b_num, time_embed, cls_token_swap, ln_t_w, ln_t_b, w_qkv, w_proj, b_proj, w_tfc, b_tfc):
    raise NotImplementedError("write your pallas kernel here")


# speedup vs baseline: 1.6322x; 1.6322x over previous
import jax
import jax.numpy as jnp
from jax.experimental import pallas as pl
from jax.experimental.pallas import tpu as pltpu

_EPS = 1e-5
_B = 8
_T = 20
_H = 224
_W = 224
_C = 1536
_NH = 12
_HD = 128
_NG = 5
_K = 5


def _pool_kernel(bb_ref, x_ref, o_ref):
    b = pl.program_id(0)
    t = pl.program_id(1)
    base = (b * _T + t) * 4
    x1 = bb_ref[base]
    y1 = bb_ref[base + 1]
    x2 = bb_ref[base + 2]
    y2 = bb_ref[base + 3]
    ri = jax.lax.broadcasted_iota(jnp.int32, (_H, _W), 0)
    ci = jax.lax.broadcasted_iota(jnp.int32, (_H, _W), 1)
    mask = (ri >= y1) & (ri < y2) & (ci >= x1) & (ci < x2)
    area = ((x2 - x1) * (y2 - y1)).astype(jnp.float32)
    inv_area = 1.0 / area
    inv_hw = 1.0 / float(_H * _W)
    for c in range(3):
        img = x_ref[0, c, 0]
        tot = jnp.sum(img, keepdims=True)
        crop = jnp.sum(jnp.where(mask, img, 0.0), keepdims=True)
        val = tot * inv_hw + crop * inv_area
        o_ref[0, 0, c] = jnp.broadcast_to(val, (1, 128))[0]


def _xt_kernel(f_ref, wr_ref, brd_ref, te_ref, ts_ref, wn_ref, bn_ref,
               xt_ref, pooled_ref):
    i = pl.program_id(0)
    xm = jax.lax.dot_general(
        f_ref[...], wr_ref[...], (((1,), (1,)), ((), ())),
        preferred_element_type=jnp.float32)
    ts = ts_ref[pl.ds(i * 4, 4), :][:, :1]
    tsadd = ts * wn_ref[...] + bn_ref[...]
    x3 = (xm.reshape(4, _T, _C) + te_ref[...][None, :, :]
          + brd_ref[...][None, :, :] + tsadd[:, None, :])
    xt_ref[...] = x3.reshape(4 * _T, _C)
    pooled_ref[0] = (x3[:, _T - 2, :] + x3[:, _T - 1, :]) * (1.0 / 3.0)


def _attn_kernel(xt_ref, cls_ref, lnw_ref, lnb_ref, wqkv_ref, wproj_ref,
                 bp_ref, wtfc_ref, bt_ref, g_ref, cl_ref, gr_ref):
    nb = 4
    ng = nb * _NG
    ncls = ng
    ngr = ng * 4
    ntok = ncls + ngr

    gr_ref[...] = xt_ref[...]
    cl_ref[...] = jnp.broadcast_to(cls_ref[...][None, :, :],
                                   (nb, _NG, _C)).reshape(ncls, _C)

    ri = jax.lax.broadcasted_iota(jnp.int32, (ntok, ntok), 0)
    ci = jax.lax.broadcasted_iota(jnp.int32, (ntok, ntok), 1)
    gid_r = jnp.where(ri < ncls, ri, jax.lax.div(ri - ncls, 4))
    gid_c = jnp.where(ci < ncls, ci, jax.lax.div(ci - ncls, 4))
    mf = (gid_r == gid_c).astype(jnp.float32)

    scale = float(_HD) ** -0.5
    lnw = lnw_ref[...]
    lnb = lnb_ref[...]
    ones_col = jnp.ones((ntok, 1), jnp.float32)
    for _ in range(4):
        tok = jnp.concatenate([cl_ref[...], gr_ref[...]], axis=0)
        mu = jnp.mean(tok, axis=1, keepdims=True)
        d = tok - mu
        var = jnp.mean(d * d, axis=1, keepdims=True)
        h = d * jax.lax.rsqrt(var + _EPS) * lnw + lnb
        qkv = jax.lax.dot_general(
            h, wqkv_ref[...], (((1,), (1,)), ((), ())),
            preferred_element_type=jnp.float32)
        oparts = []
        for hh in range(_NH):
            qh = qkv[:, hh * _HD:(hh + 1) * _HD]
            kh = qkv[:, _C + hh * _HD:_C + (hh + 1) * _HD]
            vh = qkv[:, 2 * _C + hh * _HD:2 * _C + (hh + 1) * _HD]
            s = jax.lax.dot_general(
                qh, kh, (((1,), (1,)), ((), ())),
                preferred_element_type=jnp.float32)
            e = jnp.exp(s * scale) * mf
            vaug = jnp.concatenate([vh, ones_col], axis=1)
            oa = jax.lax.dot_general(
                e, vaug, (((1,), (0,)), ((), ())),
                preferred_element_type=jnp.float32)
            oparts.append(oa[:, :_HD] * (1.0 / oa[:, _HD:_HD + 1]))
        o = jnp.concatenate(oparts, axis=1)
        p = jax.lax.dot_general(
            o, wproj_ref[...], (((1,), (1,)), ((), ())),
            preferred_element_type=jnp.float32) + bp_ref[...]
        u = jax.lax.dot_general(
            p, wtfc_ref[...], (((1,), (1,)), ((), ())),
            preferred_element_type=jnp.float32) + bt_ref[...]
        upd = u + tok
        cl_new = upd[:ncls].reshape(nb, _NG, _C)
        cl_ref[...] = jnp.concatenate(
            [cl_new[:, _NG - 1:_NG, :], cl_new[:, :_NG - 1, :]],
            axis=1).reshape(ncls, _C)
        gr_ref[...] = upd[ncls:]
    g_ref[:, :, 0:1, :] = cl_ref[...].reshape(nb, _NG, 1, _C)
    g_ref[:, :, 1:5, :] = gr_ref[...].reshape(nb, _NG, 4, _C)


def kernel(x, timestamp, bboxes, feats, w_reduce, b_reduce, w_num, b_num,
           time_embed, cls_token_swap, ln_t_w, ln_t_b, w_qkv, w_proj, b_proj,
           w_tfc, b_tfc):
    bb_flat = bboxes.reshape(-1).astype(jnp.int32)
    pool_out = pl.pallas_call(
        _pool_kernel,
        out_shape=jax.ShapeDtypeStruct((_B, _T, 3, 128), jnp.float32),
        grid_spec=pltpu.PrefetchScalarGridSpec(
            num_scalar_prefetch=1,
            grid=(_B, _T),
            in_specs=[pl.BlockSpec((1, 3, 1, _H, _W),
                                   lambda b, t, bb: (b, 0, t, 0, 0))],
            out_specs=pl.BlockSpec((1, 1, 3, 128),
                                   lambda b, t, bb: (b, t, 0, 0)),
        ),
        compiler_params=pltpu.CompilerParams(
            dimension_semantics=("parallel", "arbitrary"),
        ),
        name="bbox_pool",
    )(bb_flat, x)
    ssm_q = pool_out[:, :, :, 0]

    feats2 = feats.reshape(_B * _T, _C)
    ts2 = jnp.broadcast_to(timestamp[:, None], (_B, 128))
    brd = b_reduce.reshape(1, _C)
    bnm = b_num.reshape(1, _C)
    wnr = w_num.reshape(1, _C)
    te = time_embed.reshape(_T, _C)
    xt2, pooled = pl.pallas_call(
        _xt_kernel,
        out_shape=[jax.ShapeDtypeStruct((_B * _T, _C), jnp.float32),
                   jax.ShapeDtypeStruct((2, _B // 2, _C), jnp.float32)],
        grid=(2,),
        in_specs=[
            pl.BlockSpec((_B * _T // 2, _C), lambda i: (i, 0)),
            pl.BlockSpec(memory_space=pltpu.VMEM),
            pl.BlockSpec(memory_space=pltpu.VMEM),
            pl.BlockSpec(memory_space=pltpu.VMEM),
            pl.BlockSpec(memory_space=pltpu.VMEM),
            pl.BlockSpec(memory_space=pltpu.VMEM),
            pl.BlockSpec(memory_space=pltpu.VMEM),
        ],
        out_specs=[pl.BlockSpec((_B * _T // 2, _C), lambda i: (i, 0)),
                   pl.BlockSpec((1, _B // 2, _C), lambda i: (i, 0, 0))],
        compiler_params=pltpu.CompilerParams(
            dimension_semantics=("parallel",),
            vmem_limit_bytes=40 * 1024 * 1024,
        ),
        name="reduce_dim_xt",
    )(feats2, w_reduce, brd, te, ts2, wnr, bnm)
    pooled = pooled.reshape(_B, _C)

    cls2 = cls_token_swap.reshape(_NG, _C)
    lnw = ln_t_w.reshape(1, _C)
    lnb = ln_t_b.reshape(1, _C)
    bpj = b_proj.reshape(1, _C)
    btf = b_tfc.reshape(1, _C)
    g = pl.pallas_call(
        _attn_kernel,
        out_shape=jax.ShapeDtypeStruct((_B, _NG, _K, _C), jnp.float32),
        grid=(2,),
        in_specs=[
            pl.BlockSpec((_B * _T // 2, _C), lambda i: (i, 0)),
            pl.BlockSpec(memory_space=pltpu.VMEM),
            pl.BlockSpec(memory_space=pltpu.VMEM),
            pl.BlockSpec(memory_space=pltpu.VMEM),
            pl.BlockSpec(memory_space=pltpu.VMEM),
            pl.BlockSpec(memory_space=pltpu.VMEM),
            pl.BlockSpec(memory_space=pltpu.VMEM),
            pl.BlockSpec(memory_space=pltpu.VMEM),
            pl.BlockSpec(memory_space=pltpu.VMEM),
        ],
        out_specs=pl.BlockSpec((_B // 2, _NG, _K, _C), lambda i: (i, 0, 0, 0)),
        scratch_shapes=[
            pltpu.VMEM((_B * _NG // 2, _C), jnp.float32),
            pltpu.VMEM((_B * _T // 2, _C), jnp.float32),
        ],
        compiler_params=pltpu.CompilerParams(
            dimension_semantics=("parallel",),
            vmem_limit_bytes=56 * 1024 * 1024,
        ),
        name="group_attn",
    )(xt2, cls2, lnw, lnb, w_qkv, w_proj, bpj, w_tfc, btf)

    return ssm_q, g, pooled


# speedup vs baseline: 3.0936x; 1.8953x over previous
import jax
import jax.numpy as jnp
from jax.experimental import pallas as pl
from jax.experimental.pallas import tpu as pltpu

_EPS = 1e-5
_B = 8
_T = 20
_H = 224
_W = 224
_C = 1536
_NH = 12
_HD = 128
_NG = 5
_K = 5
_TB = 4
_NP = _B * (_T // _TB)
_NCQ = 36
_NCS = 12


def _mega_kernel(bb_ref, x_ref, f_ref, wr_ref, brd_ref, te_ref, ts_ref,
                 wn_ref, bn_ref, cls_ref, lnw_ref, lnb_ref, wqkv_ref,
                 wproj_ref, bp_ref, wtfc_ref, bt_ref,
                 o_ref, g_ref, pooled_ref,
                 wqkv_bf, wproj_bf, wtfc_bf, xt_s, cl_ref, gr_ref):
    i = pl.program_id(0)

    @pl.when(i < _NP)
    def _pool():
        b = i // (_T // _TB)
        tb = jax.lax.rem(i, _T // _TB)
        ri = jax.lax.broadcasted_iota(jnp.int32, (_H, _W), 0)
        ci = jax.lax.broadcasted_iota(jnp.int32, (_H, _W), 1)
        inv_hw = 1.0 / float(_H * _W)
        for tt in range(_TB):
            base = (b * _T + tb * _TB + tt) * 4
            x1 = bb_ref[base]
            y1 = bb_ref[base + 1]
            x2 = bb_ref[base + 2]
            y2 = bb_ref[base + 3]
            mask = (ri >= y1) & (ri < y2) & (ci >= x1) & (ci < x2)
            area = ((x2 - x1) * (y2 - y1)).astype(jnp.float32)
            inv_area = 1.0 / area
            for c in range(3):
                img = x_ref[0, c, tt]
                tot = jnp.sum(img, keepdims=True)
                crop = jnp.sum(jnp.where(mask, img, 0.0), keepdims=True)
                val = tot * inv_hw + crop * inv_area
                o_ref[0, tt, c] = jnp.broadcast_to(val, (1, 128))[0]

    @pl.when(i < _NCQ)
    def _stream_qkv():
        r = pl.multiple_of(i * _HD, _HD)
        wqkv_bf[pl.ds(r, _HD), :] = wqkv_ref[...].astype(jnp.bfloat16)

    @pl.when(i < _NCS)
    def _stream_sq():
        r = pl.multiple_of(i * _HD, _HD)
        wproj_bf[pl.ds(r, _HD), :] = wproj_ref[...].astype(jnp.bfloat16)
        wtfc_bf[pl.ds(r, _HD), :] = wtfc_ref[...].astype(jnp.bfloat16)
        xt_s[:, pl.ds(r, _HD)] = jax.lax.dot_general(
            f_ref[...], wr_ref[...], (((1,), (1,)), ((), ())),
            preferred_element_type=jnp.float32)

    @pl.when(i == _NP)
    def _attn():
        ts = ts_ref[...][:, :1]
        tsadd = ts * wn_ref[...] + bn_ref[...]
        x3 = (xt_s[...].reshape(_B, _T, _C) + te_ref[...][None, :, :]
              + brd_ref[...][None, :, :] + tsadd[:, None, :])
        pooled_ref[...] = (x3[:, _T - 2, :] + x3[:, _T - 1, :]) * (1.0 / 3.0)
        gr_ref[...] = x3.reshape(_B * _T, _C)
        ncls = _B * _NG
        ntok = _B * _T + ncls
        cl_ref[...] = jnp.broadcast_to(cls_ref[...][None, :, :],
                                       (_B, _NG, _C)).reshape(ncls, _C)
        ri = jax.lax.broadcasted_iota(jnp.int32, (ntok, ntok), 0)
        ci = jax.lax.broadcasted_iota(jnp.int32, (ntok, ntok), 1)
        gid_r = jnp.where(ri < ncls, ri, jax.lax.div(ri - ncls, 4))
        gid_c = jnp.where(ci < ncls, ci, jax.lax.div(ci - ncls, 4))
        mf = (gid_r == gid_c).astype(jnp.float32)
        scale = float(_HD) ** -0.5
        lnw = lnw_ref[...]
        lnb = lnb_ref[...]
        ones_col = jnp.ones((ntok, 1), jnp.float32)
        for _ in range(4):
            tok = jnp.concatenate([cl_ref[...], gr_ref[...]], axis=0)
            mu = jnp.mean(tok, axis=1, keepdims=True)
            d = tok - mu
            var = jnp.mean(d * d, axis=1, keepdims=True)
            h = ((d * jax.lax.rsqrt(var + _EPS)) * lnw + lnb)
            qkv = jax.lax.dot_general(
                h.astype(jnp.bfloat16), wqkv_bf[...], (((1,), (1,)), ((), ())),
                preferred_element_type=jnp.float32)
            oparts = []
            for hh in range(_NH):
                qh = qkv[:, hh * _HD:(hh + 1) * _HD]
                kh = qkv[:, _C + hh * _HD:_C + (hh + 1) * _HD]
                vh = qkv[:, 2 * _C + hh * _HD:2 * _C + (hh + 1) * _HD]
                s = jax.lax.dot_general(
                    qh, kh, (((1,), (1,)), ((), ())),
                    preferred_element_type=jnp.float32)
                e = jnp.exp(s * scale) * mf
                vaug = jnp.concatenate([vh, ones_col], axis=1)
                oa = jax.lax.dot_general(
                    e, vaug, (((1,), (0,)), ((), ())),
                    preferred_element_type=jnp.float32)
                oparts.append(oa[:, :_HD] * (1.0 / oa[:, _HD:_HD + 1]))
            o = jnp.concatenate(oparts, axis=1)
            p = jax.lax.dot_general(
                o.astype(jnp.bfloat16), wproj_bf[...], (((1,), (1,)), ((), ())),
                preferred_element_type=jnp.float32) + bp_ref[...]
            u = jax.lax.dot_general(
                p.astype(jnp.bfloat16), wtfc_bf[...], (((1,), (1,)), ((), ())),
                preferred_element_type=jnp.float32) + bt_ref[...]
            upd = u + tok
            cl_new = upd[:ncls].reshape(_B, _NG, _C)
            cl_ref[...] = jnp.concatenate(
                [cl_new[:, _NG - 1:_NG, :], cl_new[:, :_NG - 1, :]],
                axis=1).reshape(ncls, _C)
            gr_ref[...] = upd[ncls:]
        g_ref[:, :, 0:1, :] = cl_ref[...].reshape(_B, _NG, 1, _C)
        g_ref[:, :, 1:5, :] = gr_ref[...].reshape(_B, _NG, 4, _C)


def kernel(x, timestamp, bboxes, feats, w_reduce, b_reduce, w_num, b_num,
           time_embed, cls_token_swap, ln_t_w, ln_t_b, w_qkv, w_proj, b_proj,
           w_tfc, b_tfc):
    bb_flat = bboxes.reshape(-1).astype(jnp.int32)
    feats2 = feats.reshape(_B * _T, _C)
    ts2 = jnp.broadcast_to(timestamp[:, None], (_B, 128))
    brd = b_reduce.reshape(1, _C)
    bnm = b_num.reshape(1, _C)
    wnr = w_num.reshape(1, _C)
    te = time_embed.reshape(_T, _C)
    cls2 = cls_token_swap.reshape(_NG, _C)
    lnw = ln_t_w.reshape(1, _C)
    lnb = ln_t_b.reshape(1, _C)
    bpj = b_proj.reshape(1, _C)
    btf = b_tfc.reshape(1, _C)

    npool = _T // _TB
    pool_out, g, pooled = pl.pallas_call(
        _mega_kernel,
        out_shape=[jax.ShapeDtypeStruct((_B, _T, 3, 128), jnp.float32),
                   jax.ShapeDtypeStruct((_B, _NG, _K, _C), jnp.float32),
                   jax.ShapeDtypeStruct((_B, _C), jnp.float32)],
        grid_spec=pltpu.PrefetchScalarGridSpec(
            num_scalar_prefetch=1,
            grid=(_NP + 1,),
            in_specs=[
                pl.BlockSpec((1, 3, _TB, _H, _W),
                             lambda i, bb: (jnp.minimum(i, _NP - 1) // npool, 0,
                                            jax.lax.rem(jnp.minimum(i, _NP - 1),
                                                        npool), 0, 0)),
                pl.BlockSpec(memory_space=pltpu.VMEM),
                pl.BlockSpec((_HD, _C),
                             lambda i, bb: (jnp.minimum(i, _NCS - 1), 0)),
                pl.BlockSpec(memory_space=pltpu.VMEM),
                pl.BlockSpec(memory_space=pltpu.VMEM),
                pl.BlockSpec(memory_space=pltpu.VMEM),
                pl.BlockSpec(memory_space=pltpu.VMEM),
                pl.BlockSpec(memory_space=pltpu.VMEM),
                pl.BlockSpec(memory_space=pltpu.VMEM),
                pl.BlockSpec(memory_space=pltpu.VMEM),
                pl.BlockSpec(memory_space=pltpu.VMEM),
                pl.BlockSpec((_HD, _C),
                             lambda i, bb: (jnp.minimum(i, _NCQ - 1), 0)),
                pl.BlockSpec((_HD, _C),
                             lambda i, bb: (jnp.minimum(i, _NCS - 1), 0)),
                pl.BlockSpec(memory_space=pltpu.VMEM),
                pl.BlockSpec((_HD, _C),
                             lambda i, bb: (jnp.minimum(i, _NCS - 1), 0)),
                pl.BlockSpec(memory_space=pltpu.VMEM),
            ],
            out_specs=[
                pl.BlockSpec((1, _TB, 3, 128),
                             lambda i, bb: (jnp.minimum(i, _NP - 1) // npool,
                                            jax.lax.rem(jnp.minimum(i, _NP - 1),
                                                        npool), 0, 0)),
                pl.BlockSpec(memory_space=pltpu.VMEM),
                pl.BlockSpec(memory_space=pltpu.VMEM),
            ],
            scratch_shapes=[
                pltpu.VMEM((3 * _C, _C), jnp.bfloat16),
                pltpu.VMEM((_C, _C), jnp.bfloat16),
                pltpu.VMEM((_C, _C), jnp.bfloat16),
                pltpu.VMEM((_B * _T, _C), jnp.float32),
                pltpu.VMEM((_B * _NG, _C), jnp.float32),
                pltpu.VMEM((_B * _T, _C), jnp.float32),
            ],
        ),
        compiler_params=pltpu.CompilerParams(
            dimension_semantics=("arbitrary",),
            vmem_limit_bytes=52 * 1024 * 1024,
        ),
        name="fused_vit",
    )(bb_flat, x, feats2, w_reduce, brd, te, ts2, wnr, bnm, cls2, lnw, lnb,
      w_qkv, w_proj, bpj, w_tfc, btf)

    return pool_out[:, :, :, 0], g, pooled
